# baseline (speedup 1.0000x reference)
import functools

import jax
import jax.numpy as jnp
from jax import lax
from jax.experimental import pallas as pl
from jax.experimental.pallas import tpu as pltpu
from jax.experimental.pallas import tpu_sc as plsc

B = 65536
D = 256
V = 8193
VP = 8704
L = 16
NC, NS = 2, 16
NW = NC * NS
RPW = B // NW
CH = 64
NCHUNK = RPW // CH
DW = D // 2
LG = D // L
HI_MASK = -65536


def _sc_partial_sums(q4, tab_packed, idx3):
    mesh = plsc.VectorSubcoreMesh(core_axis_name="c", subcore_axis_name="s")

    @functools.partial(
        pl.kernel,
        mesh=mesh,
        out_type=jax.ShapeDtypeStruct((NW, L), jnp.float32),
        scratch_types=[
            pltpu.VMEM((NCHUNK, CH), jnp.int32),
            pltpu.VMEM((CH, DW), jnp.int32),
            pltpu.VMEM((CH, DW), jnp.int32),
            pltpu.VMEM((CH, D), jnp.float32),
            pltpu.VMEM((CH, D), jnp.float32),
            pltpu.VMEM((L,), jnp.float32),
            pltpu.SemaphoreType.DMA,
            pltpu.SemaphoreType.DMA,
        ],
    )
    def k(q_hbm, tab_hbm, idx_hbm, out_hbm,
          idx_v, rows0, rows1, q0, q1, acc_v, sem0, sem1):
        wid = lax.axis_index("s") * NC + lax.axis_index("c")
        pltpu.sync_copy(idx_hbm.at[wid], idx_v)

        bufs = ((rows0, q0, sem0), (rows1, q1, sem1))

        def start(c, b):
            rows_b, q_b, sem_b = bufs[b]
            pltpu.async_copy(tab_hbm.at[idx_v.at[c]], rows_b, sem_b)
            pltpu.async_copy(q_hbm.at[wid, c], q_b, sem_b)

        def wait_and_compute(c, b, accs):
            rows_b, q_b, sem_b = bufs[b]
            pltpu.make_async_copy(tab_hbm.at[pl.ds(0, CH)], rows_b, sem_b).wait()
            pltpu.make_async_copy(q_hbm.at[wid, c], q_b, sem_b).wait()

            def row_body(i, accs):
                out = []
                for j in range(LG // 2):
                    v32 = rows_b[i, pl.ds(L * j, L)]
                    lo = jax.lax.bitcast_convert_type(v32 << 16, jnp.float32)
                    hi = jax.lax.bitcast_convert_type(v32 & jnp.int32(HI_MASK), jnp.float32)
                    dlo = lo - q_b[i, pl.ds(32 * j, L)]
                    dhi = hi - q_b[i, pl.ds(32 * j + L, L)]
                    out.append(accs[2 * j] + dlo * dlo)
                    out.append(accs[2 * j + 1] + dhi * dhi)
                return tuple(out)

            return lax.fori_loop(0, CH, row_body, accs)

        start(0, 0)
        start(1, 1)
        zero = jnp.zeros((L,), jnp.float32)

        def outer(i, accs):
            c0 = 2 * i
            accs = wait_and_compute(c0, 0, accs)

            @pl.when(c0 + 2 < NCHUNK)
            def _():
                start(c0 + 2, 0)

            accs = wait_and_compute(c0 + 1, 1, accs)

            @pl.when(c0 + 3 < NCHUNK)
            def _():
                start(c0 + 3, 1)

            return accs

        accs = lax.fori_loop(0, NCHUNK // 2, outer, (zero,) * LG)

        total = accs[0]
        for j in range(1, LG):
            total = total + accs[j]
        acc_v[...] = total
        pltpu.sync_copy(acc_v, out_hbm.at[wid])

    return k(q4, tab_packed, idx3)


def _pack_codebook(class_embeddings):
    cb = jnp.pad(class_embeddings.astype(jnp.bfloat16), ((0, VP - V), (0, 0)))
    cb = cb.reshape(VP, LG // 2, 2, L).transpose(0, 1, 3, 2)
    return jax.lax.bitcast_convert_type(cb, jnp.int32).reshape(VP, DW)


def kernel(query_features, class_embeddings, class_indices):
    q4 = query_features.reshape(NW, NCHUNK, CH, D)
    idx3 = class_indices.astype(jnp.int32).reshape(NW, NCHUNK, CH)
    tab_packed = _pack_codebook(class_embeddings)
    partial = _sc_partial_sums(q4, tab_packed, idx3)
    return jnp.sum(partial) * (1.25 / (B * D))


# speedup vs baseline: 1.2481x; 1.2481x over previous
import functools

import jax
import jax.numpy as jnp
from jax import lax
from jax.experimental import pallas as pl
from jax.experimental.pallas import tpu as pltpu
from jax.experimental.pallas import tpu_sc as plsc

B = 65536
D = 256
V = 8193
L = 16
NC, NS = 2, 16
NW = NC * NS
RPW = B // NW
CH = 64
NCHUNK = RPW // CH
DW = D // 2
LG = D // L
HI_MASK = -65536


def _tc_pack_codebook(c):
    def body(x_ref, o_ref):
        u = jax.lax.bitcast_convert_type(x_ref[...], jnp.uint32)
        r = (u + jnp.uint32(0x7FFF) + ((u >> 16) & jnp.uint32(1))) >> 16
        lo = r[:, :DW]
        hi = r[:, DW:]
        o_ref[...] = jax.lax.bitcast_convert_type(lo | (hi << 16), jnp.int32)

    return pl.pallas_call(
        body, out_shape=jax.ShapeDtypeStruct((V, DW), jnp.int32))(c)


def _sc_partial_sums(q4, tab_packed, idx3):
    mesh = plsc.VectorSubcoreMesh(core_axis_name="c", subcore_axis_name="s")

    @functools.partial(
        pl.kernel,
        mesh=mesh,
        out_type=jax.ShapeDtypeStruct((NW, L), jnp.float32),
        scratch_types=[
            pltpu.VMEM((NCHUNK, CH), jnp.int32),
            pltpu.VMEM((CH, DW), jnp.int32),
            pltpu.VMEM((CH, DW), jnp.int32),
            pltpu.VMEM((CH, D), jnp.float32),
            pltpu.VMEM((CH, D), jnp.float32),
            pltpu.VMEM((L,), jnp.float32),
            pltpu.SemaphoreType.DMA,
            pltpu.SemaphoreType.DMA,
        ],
    )
    def k(q_hbm, tab_hbm, idx_hbm, out_hbm,
          idx_v, rows0, rows1, q0, q1, acc_v, sem0, sem1):
        wid = lax.axis_index("s") * NC + lax.axis_index("c")
        pltpu.sync_copy(idx_hbm.at[wid], idx_v)

        bufs = ((rows0, q0, sem0), (rows1, q1, sem1))

        def start(c, b):
            rows_b, q_b, sem_b = bufs[b]
            pltpu.async_copy(tab_hbm.at[idx_v.at[c]], rows_b, sem_b)
            pltpu.async_copy(q_hbm.at[wid, c], q_b, sem_b)

        def wait_and_compute(c, b, accs):
            rows_b, q_b, sem_b = bufs[b]
            pltpu.make_async_copy(tab_hbm.at[pl.ds(0, CH)], rows_b, sem_b).wait()
            pltpu.make_async_copy(q_hbm.at[wid, c], q_b, sem_b).wait()

            def row_body(i, accs):
                out = [None] * LG
                for j in range(LG // 2):
                    v32 = rows_b[i, pl.ds(L * j, L)]
                    lo = jax.lax.bitcast_convert_type(v32 << 16, jnp.float32)
                    hi = jax.lax.bitcast_convert_type(
                        v32 & jnp.int32(HI_MASK), jnp.float32)
                    dlo = lo - q_b[i, pl.ds(L * j, L)]
                    dhi = hi - q_b[i, pl.ds(DW + L * j, L)]
                    out[j] = accs[j] + dlo * dlo
                    out[LG // 2 + j] = accs[LG // 2 + j] + dhi * dhi
                return tuple(out)

            return lax.fori_loop(0, CH, row_body, accs)

        start(0, 0)
        start(1, 1)
        zero = jnp.zeros((L,), jnp.float32)

        def outer(i, accs):
            c0 = 2 * i
            accs = wait_and_compute(c0, 0, accs)

            @pl.when(c0 + 2 < NCHUNK)
            def _():
                start(c0 + 2, 0)

            accs = wait_and_compute(c0 + 1, 1, accs)

            @pl.when(c0 + 3 < NCHUNK)
            def _():
                start(c0 + 3, 1)

            return accs

        accs = lax.fori_loop(0, NCHUNK // 2, outer, (zero,) * LG)

        total = accs[0]
        for j in range(1, LG):
            total = total + accs[j]
        acc_v[...] = total
        pltpu.sync_copy(acc_v, out_hbm.at[wid])

    return k(q4, tab_packed, idx3)


def kernel(query_features, class_embeddings, class_indices):
    q4 = query_features.reshape(NW, NCHUNK, CH, D)
    idx3 = class_indices.astype(jnp.int32).reshape(NW, NCHUNK, CH)
    tab_packed = _tc_pack_codebook(class_embeddings)
    partial = _sc_partial_sums(q4, tab_packed, idx3)
    return jnp.sum(partial) * (1.25 / (B * D))


# speedup vs baseline: 1.3509x; 1.0823x over previous
import functools

import jax
import jax.numpy as jnp
from jax import lax
from jax.experimental import pallas as pl
from jax.experimental.pallas import tpu as pltpu
from jax.experimental.pallas import tpu_sc as plsc

B = 65536
D = 256
V = 8193
L = 16
NC, NS = 2, 16
NW = NC * NS
RPW = B // NW
CH = 128
NCHUNK = RPW // CH
DW = D // 2
LG = D // L
HI_MASK = -65536


def _tc_pack_codebook(c):
    def body(x_ref, o_ref):
        u = jax.lax.bitcast_convert_type(x_ref[...], jnp.uint32)
        r = (u + jnp.uint32(0x7FFF) + ((u >> 16) & jnp.uint32(1))) >> 16
        lo = r[:, :DW]
        hi = r[:, DW:]
        o_ref[...] = jax.lax.bitcast_convert_type(lo | (hi << 16), jnp.int32)

    return pl.pallas_call(
        body, out_shape=jax.ShapeDtypeStruct((V, DW), jnp.int32))(c)


def _sc_partial_sums(q4, tab_packed, idx3):
    mesh = plsc.VectorSubcoreMesh(core_axis_name="c", subcore_axis_name="s")

    @functools.partial(
        pl.kernel,
        mesh=mesh,
        out_type=jax.ShapeDtypeStruct((NW, L), jnp.float32),
        scratch_types=[
            pltpu.VMEM((NCHUNK, CH), jnp.int32),
            pltpu.VMEM((CH, DW), jnp.int32),
            pltpu.VMEM((CH, DW), jnp.int32),
            pltpu.VMEM((CH, D), jnp.float32),
            pltpu.VMEM((CH, D), jnp.float32),
            pltpu.VMEM((L,), jnp.float32),
            pltpu.SemaphoreType.DMA,
            pltpu.SemaphoreType.DMA,
        ],
    )
    def k(q_hbm, tab_hbm, idx_hbm, out_hbm,
          idx_v, rows0, rows1, q0, q1, acc_v, sem0, sem1):
        wid = lax.axis_index("s") * NC + lax.axis_index("c")
        pltpu.sync_copy(idx_hbm.at[wid], idx_v)

        bufs = ((rows0, q0, sem0), (rows1, q1, sem1))

        def start(c, b):
            rows_b, q_b, sem_b = bufs[b]
            pltpu.async_copy(tab_hbm.at[idx_v.at[c]], rows_b, sem_b)
            pltpu.async_copy(q_hbm.at[wid, c], q_b, sem_b)

        def wait_and_compute(c, b, accs):
            rows_b, q_b, sem_b = bufs[b]
            pltpu.make_async_copy(tab_hbm.at[pl.ds(0, CH)], rows_b, sem_b).wait()
            pltpu.make_async_copy(q_hbm.at[wid, c], q_b, sem_b).wait()

            def row_body(i, accs):
                out = [None] * LG
                for j in range(LG // 2):
                    v32 = rows_b[i, pl.ds(L * j, L)]
                    lo = jax.lax.bitcast_convert_type(v32 << 16, jnp.float32)
                    hi = jax.lax.bitcast_convert_type(
                        v32 & jnp.int32(HI_MASK), jnp.float32)
                    dlo = lo - q_b[i, pl.ds(L * j, L)]
                    dhi = hi - q_b[i, pl.ds(DW + L * j, L)]
                    out[j] = accs[j] + dlo * dlo
                    out[LG // 2 + j] = accs[LG // 2 + j] + dhi * dhi
                return tuple(out)

            return lax.fori_loop(0, CH, row_body, accs)

        start(0, 0)
        start(1, 1)
        zero = jnp.zeros((L,), jnp.float32)

        def outer(i, accs):
            c0 = 2 * i
            accs = wait_and_compute(c0, 0, accs)

            @pl.when(c0 + 2 < NCHUNK)
            def _():
                start(c0 + 2, 0)

            accs = wait_and_compute(c0 + 1, 1, accs)

            @pl.when(c0 + 3 < NCHUNK)
            def _():
                start(c0 + 3, 1)

            return accs

        accs = lax.fori_loop(0, NCHUNK // 2, outer, (zero,) * LG)

        total = accs[0]
        for j in range(1, LG):
            total = total + accs[j]
        acc_v[...] = total
        pltpu.sync_copy(acc_v, out_hbm.at[wid])

    return k(q4, tab_packed, idx3)


def kernel(query_features, class_embeddings, class_indices):
    q4 = query_features.reshape(NW, NCHUNK, CH, D)
    idx3 = class_indices.astype(jnp.int32).reshape(NW, NCHUNK, CH)
    tab_packed = _tc_pack_codebook(class_embeddings)
    partial = _sc_partial_sums(q4, tab_packed, idx3)
    return jnp.sum(partial) * (1.25 / (B * D))
